# baseline (speedup 1.0000x reference)
import functools

import jax
import jax.numpy as jnp
from jax import lax
from jax.experimental import pallas as pl
from jax.experimental.pallas import tpu as pltpu
from jax.experimental.pallas import tpu_sc as plsc

BEAM = 16
VOCAB = 32768
SEQ = 20
HID = 512
NLAYERS = 2
LANES = 16
GROUP = 4
NGROUPS = VOCAB // (LANES * GROUP)
P1UNROLL = 16

_mesh = plsc.VectorSubcoreMesh(
    core_axis_name="c", subcore_axis_name="s", num_cores=1)


def _merge_top16(ak, ap, xk, xp):
    sk, sp = plsc.sort_key_val(xk, xp)
    bk = lax.rev(sk, (0,))
    bp = lax.rev(sp, (0,))
    take = ak >= bk
    nk = jnp.where(take, ak, bk)
    npay = jnp.where(take, ap, bp)
    return plsc.sort_key_val(nk, npay)


_OUT_TYPE = (
    jax.ShapeDtypeStruct((BEAM * SEQ,), jnp.int32),
    jax.ShapeDtypeStruct((BEAM * SEQ,), jnp.float32),
    jax.ShapeDtypeStruct((BEAM,), jnp.float32),
    jax.ShapeDtypeStruct((NLAYERS * BEAM, HID), jnp.float32),
)
_SCRATCH = [
    pltpu.VMEM((VOCAB,), jnp.float32),
    pltpu.VMEM((LANES,), jnp.float32),
    pltpu.VMEM((LANES,), jnp.int32),
    pltpu.VMEM_SHARED((BEAM * LANES,), jnp.float32),
    pltpu.VMEM_SHARED((BEAM * LANES,), jnp.int32),
    pltpu.VMEM((BEAM * LANES,), jnp.float32),
    pltpu.VMEM((BEAM * LANES,), jnp.int32),
    pltpu.VMEM((BEAM,), jnp.float32),
    pltpu.VMEM((BEAM * SEQ,), jnp.int32),
    pltpu.VMEM((BEAM * SEQ,), jnp.float32),
    pltpu.VMEM((LANES,), jnp.int32),
    pltpu.VMEM((BEAM * SEQ,), jnp.int32),
    pltpu.VMEM((BEAM * SEQ,), jnp.float32),
    pltpu.VMEM((NLAYERS * BEAM,), jnp.int32),
    pltpu.VMEM((NLAYERS * BEAM, HID), jnp.float32),
    pltpu.SemaphoreType.DMA,
    pltpu.SemaphoreType.DMA,
]


def _beam_body(lp_hbm, seq_hbm, seqlp_hbm, sum_hbm, state_hbm, tvec_hbm,
               out_seq_hbm, out_seqlp_hbm, out_sum_hbm, out_state_hbm,
               row_v, stage_f, stage_i, shared_v, shared_w,
               vals_v, words_v, sums_v, seq_v, seqlp_v, tvec_v,
               oseq_v, oseqlp_v, sidx_v, srows_v, sem, psem):
    s = lax.axis_index("s")
    iota = lax.iota(jnp.int32, LANES)
    neg = jnp.full((LANES,), -jnp.inf, jnp.float32)

    @pl.when(s == 0)
    def _prefetch():
        pltpu.async_copy(sum_hbm, sums_v, psem)
        pltpu.async_copy(seq_hbm, seq_v, psem)
        pltpu.async_copy(seqlp_hbm, seqlp_v, psem)
        pltpu.async_copy(tvec_hbm, tvec_v, psem)

    HALF = VOCAB // 2
    cp2 = pltpu.make_async_copy(
        lp_hbm.at[s, pl.ds(HALF, HALF)], row_v.at[pl.ds(HALF, HALF)], sem)
    cp2.start()
    pltpu.sync_copy(lp_hbm.at[s, pl.ds(0, HALF)], row_v.at[pl.ds(0, HALF)])

    def p1body(i, mcry):
        b = i * (LANES * P1UNROLL)
        ms = [row_v[pl.ds(b + k * LANES, LANES)] for k in range(P1UNROLL)]
        while len(ms) > 1:
            ms = [jnp.maximum(ms[j], ms[j + 1])
                  for j in range(0, len(ms) - 1, 2)] + (
                      [ms[-1]] if len(ms) % 2 else [])
        return jnp.maximum(mcry, ms[0])

    P1N = VOCAB // (LANES * P1UNROLL)
    m_half = lax.fori_loop(0, P1N // 2, p1body, neg)
    cp2.wait()
    m_all = lax.fori_loop(P1N // 2, P1N, p1body, m_half)
    s_sorted = jnp.sort(m_all)
    u = plsc.bitcast(s_sorted, jnp.int32)
    nd_bits = jnp.where(s_sorted > 0, u - 1,
                        jnp.where(s_sorted == 0, jnp.int32(-2147483647),
                                  u + 1))
    thr0 = jnp.broadcast_to(plsc.bitcast(nd_bits, jnp.float32)[0], (LANES,))

    def _tree(vs):
        vs = list(vs)
        while len(vs) > 1:
            vs = [jnp.maximum(vs[i], vs[i + 1])
                  for i in range(0, len(vs) - 1, 2)] + (
                      [vs[-1]] if len(vs) % 2 else [])
        return vs[0]

    GB = LANES * GROUP

    SUBS = 8

    def body(g, carry):
        ak, ap, thr = carry
        base = g * (SUBS * GB)
        xss = [[row_v[pl.ds(base + q * GB + k * LANES, LANES)]
                for k in range(GROUP)] for q in range(SUBS)]
        mq = [_tree(xs) for xs in xss]
        hit = jnp.any(_tree(mq) > thr)

        def sub(xs, m, sub_base, cry2):
            hs = jnp.any(m > cry2[2])

            def acc2(c3):
                ak, ap, _ = c3
                for k in range(GROUP):
                    idxv = iota + (sub_base + k * LANES)
                    ak, ap = _merge_top16(ak, ap, xs[k], idxv)
                return ak, ap, jnp.maximum(
                    jnp.broadcast_to(ak[0], (LANES,)), thr0)

            return lax.cond(hs, acc2, lambda c3: c3, cry2)

        def accept(cry):
            for h in range(SUBS // 2):
                def pair(c3, h=h):
                    def inner(c4, h=h):
                        c4 = sub(xss[2 * h], mq[2 * h],
                                 base + 2 * h * GB, c4)
                        return sub(xss[2 * h + 1], mq[2 * h + 1],
                                   base + (2 * h + 1) * GB, c4)

                    hp = jnp.any(
                        jnp.maximum(mq[2 * h], mq[2 * h + 1]) > c3[2])
                    return lax.cond(hp, inner, lambda c4: c4, c3)

                cry = pair(cry)
            return cry

        return lax.cond(hit, accept, lambda cry: cry, (ak, ap, thr))

    ak, ap, _ = lax.fori_loop(
        0, NGROUPS // SUBS, body,
        (neg, jnp.zeros((LANES,), jnp.int32), thr0))
    stage_f[...] = lax.rev(ak, (0,))
    stage_i[...] = lax.rev(ap, (0,))
    pltpu.sync_copy(stage_f, shared_v.at[pl.ds(s * LANES, LANES)])
    pltpu.sync_copy(stage_i, shared_w.at[pl.ds(s * LANES, LANES)])

    plsc.subcore_barrier()

    @pl.when(s == 0)
    def _tile0():
        pltpu.sync_copy(shared_v, vals_v)
        pltpu.sync_copy(shared_w, words_v)
        pltpu.make_async_copy(sum_hbm, sums_v, psem).wait()
        pltpu.make_async_copy(seq_hbm, seq_v, psem).wait()
        pltpu.make_async_copy(seqlp_hbm, seqlp_v, psem).wait()
        pltpu.make_async_copy(tvec_hbm, tvec_v, psem).wait()
        sumvec = sums_v[...]
        tvec = tvec_v[...]

        cols = []
        for cc in range(LANES):
            colv = plsc.load_gather(vals_v, [iota * LANES + cc])
            sk, sp = plsc.sort_key_val(colv + sumvec, iota + cc * LANES)
            cols.append((sk, sp))

        def merge_sorted(a, b):
            bk = lax.rev(b[0], (0,))
            bp = lax.rev(b[1], (0,))
            take = a[0] >= bk
            nk = jnp.where(take, a[0], bk)
            npay = jnp.where(take, a[1], bp)
            sk, sp = plsc.sort_key_val(nk, npay)
            return sk, sp

        while len(cols) > 1:
            cols = [merge_sorted(cols[i], cols[i + 1])
                    for i in range(0, len(cols), 2)]
        ak, ap = cols[0]
        fk = lax.rev(ak, (0,))
        fp = lax.rev(ap, (0,))
        r_sel = jnp.bitwise_and(fp, LANES - 1)
        c_sel = lax.shift_right_logical(fp, 4)
        words = plsc.load_gather(words_v, [r_sel * LANES + c_sel])
        wprob = plsc.load_gather(vals_v, [r_sel * LANES + c_sel])

        stage_f[...] = fk
        pltpu.async_copy(stage_f, out_sum_hbm, psem)

        for j in range(SEQ):
            jfull = jnp.full((LANES,), j, jnp.int32)
            am = jfull < tvec
            bm = jfull == tvec
            rows = jnp.where(am, r_sel, iota)
            colseq = plsc.load_gather(seq_v, [rows * SEQ + j])
            colseq = jnp.where(bm, words, colseq)
            plsc.store_scatter(oseq_v, [iota * SEQ + j], colseq)
            collp = plsc.load_gather(seqlp_v, [rows * SEQ + j])
            collp = jnp.where(bm, wprob, collp)
            plsc.store_scatter(oseqlp_v, [iota * SEQ + j], collp)
        pltpu.async_copy(oseq_v, out_seq_hbm, psem)
        pltpu.async_copy(oseqlp_v, out_seqlp_hbm, psem)

        sidx_v[pl.ds(0, LANES)] = r_sel
        sidx_v[pl.ds(LANES, LANES)] = r_sel + LANES
        pltpu.async_copy(state_hbm.at[sidx_v], srows_v, sem).wait()
        pltpu.sync_copy(srows_v, out_state_hbm)
        pltpu.make_async_copy(stage_f, out_sum_hbm, psem).wait()
        pltpu.make_async_copy(oseq_v, out_seq_hbm, psem).wait()
        pltpu.make_async_copy(oseqlp_v, out_seqlp_hbm, psem).wait()


_beam_step = pl.kernel(
    _beam_body,
    out_type=_OUT_TYPE,
    mesh=_mesh,
    scratch_types=_SCRATCH,
    compiler_params=pltpu.CompilerParams(needs_layout_passes=False),
)


def kernel(logprobsf, beam_seq, beam_seq_logprobs, beam_logprobs_sum, state,
           beam_size, t):
    del beam_size
    state2d = state.reshape(NLAYERS * BEAM, HID)
    tvec = jnp.broadcast_to(jnp.asarray(t, jnp.int32), (LANES,))
    oseq, oseqlp, osum, ostate = _beam_step(
        logprobsf, beam_seq.reshape(-1), beam_seq_logprobs.reshape(-1),
        beam_logprobs_sum, state2d, tvec)
    return (oseq.reshape(BEAM, SEQ), oseqlp.reshape(BEAM, SEQ), osum,
            ostate.reshape(NLAYERS, BEAM, HID))


# speedup vs baseline: 1.0459x; 1.0459x over previous
import functools

import jax
import jax.numpy as jnp
from jax import lax
from jax.experimental import pallas as pl
from jax.experimental.pallas import tpu as pltpu
from jax.experimental.pallas import tpu_sc as plsc

BEAM = 16
VOCAB = 32768
SEQ = 20
HID = 512
NLAYERS = 2
LANES = 16
GROUP = 8
NGROUPS = VOCAB // (LANES * GROUP)
P1UNROLL = 16

_mesh = plsc.VectorSubcoreMesh(
    core_axis_name="c", subcore_axis_name="s", num_cores=1)


def _merge_top16(ak, ap, xk, xp):
    sk, sp = plsc.sort_key_val(xk, xp)
    bk = lax.rev(sk, (0,))
    bp = lax.rev(sp, (0,))
    take = ak >= bk
    nk = jnp.where(take, ak, bk)
    npay = jnp.where(take, ap, bp)
    return plsc.sort_key_val(nk, npay)


_OUT_TYPE = (
    jax.ShapeDtypeStruct((BEAM * SEQ,), jnp.int32),
    jax.ShapeDtypeStruct((BEAM * SEQ,), jnp.float32),
    jax.ShapeDtypeStruct((BEAM,), jnp.float32),
    jax.ShapeDtypeStruct((NLAYERS * BEAM, HID), jnp.float32),
)
_SCRATCH = [
    pltpu.VMEM((VOCAB,), jnp.float32),
    pltpu.VMEM((LANES,), jnp.float32),
    pltpu.VMEM((LANES,), jnp.int32),
    pltpu.VMEM_SHARED((BEAM * LANES,), jnp.float32),
    pltpu.VMEM_SHARED((BEAM * LANES,), jnp.int32),
    pltpu.VMEM((BEAM * LANES,), jnp.float32),
    pltpu.VMEM((BEAM * LANES,), jnp.int32),
    pltpu.VMEM((BEAM,), jnp.float32),
    pltpu.VMEM((BEAM * SEQ,), jnp.int32),
    pltpu.VMEM((BEAM * SEQ,), jnp.float32),
    pltpu.VMEM((LANES,), jnp.int32),
    pltpu.VMEM((BEAM * SEQ,), jnp.int32),
    pltpu.VMEM((BEAM * SEQ,), jnp.float32),
    pltpu.VMEM((NLAYERS * BEAM,), jnp.int32),
    pltpu.VMEM((NLAYERS * BEAM, HID), jnp.float32),
    pltpu.SemaphoreType.DMA,
    pltpu.SemaphoreType.DMA,
]


def _beam_body(lp_hbm, seq_hbm, seqlp_hbm, sum_hbm, state_hbm, tvec_hbm,
               out_seq_hbm, out_seqlp_hbm, out_sum_hbm, out_state_hbm,
               row_v, stage_f, stage_i, shared_v, shared_w,
               vals_v, words_v, sums_v, seq_v, seqlp_v, tvec_v,
               oseq_v, oseqlp_v, sidx_v, srows_v, sem, psem):
    s = lax.axis_index("s")
    iota = lax.iota(jnp.int32, LANES)
    neg = jnp.full((LANES,), -jnp.inf, jnp.float32)

    @pl.when(s == 0)
    def _prefetch():
        pltpu.async_copy(sum_hbm, sums_v, psem)
        pltpu.async_copy(seq_hbm, seq_v, psem)
        pltpu.async_copy(seqlp_hbm, seqlp_v, psem)
        pltpu.async_copy(tvec_hbm, tvec_v, psem)

    HALF = VOCAB // 2
    cp2 = pltpu.make_async_copy(
        lp_hbm.at[s, pl.ds(HALF, HALF)], row_v.at[pl.ds(HALF, HALF)], sem)
    cp2.start()
    pltpu.sync_copy(lp_hbm.at[s, pl.ds(0, HALF)], row_v.at[pl.ds(0, HALF)])

    def p1body(i, mcry):
        b = i * (LANES * P1UNROLL)
        ms = [row_v[pl.ds(b + k * LANES, LANES)] for k in range(P1UNROLL)]
        while len(ms) > 1:
            ms = [jnp.maximum(ms[j], ms[j + 1])
                  for j in range(0, len(ms) - 1, 2)] + (
                      [ms[-1]] if len(ms) % 2 else [])
        return jnp.maximum(mcry, ms[0])

    P1N = VOCAB // (LANES * P1UNROLL)
    m_half = lax.fori_loop(0, P1N // 2, p1body, neg)
    cp2.wait()
    m_all = lax.fori_loop(P1N // 2, P1N, p1body, m_half)
    s_sorted = jnp.sort(m_all)
    u = plsc.bitcast(s_sorted, jnp.int32)
    nd_bits = jnp.where(s_sorted > 0, u - 1,
                        jnp.where(s_sorted == 0, jnp.int32(-2147483647),
                                  u + 1))
    thr0 = jnp.broadcast_to(plsc.bitcast(nd_bits, jnp.float32)[0], (LANES,))

    def _tree(vs):
        vs = list(vs)
        while len(vs) > 1:
            vs = [jnp.maximum(vs[i], vs[i + 1])
                  for i in range(0, len(vs) - 1, 2)] + (
                      [vs[-1]] if len(vs) % 2 else [])
        return vs[0]

    GB = LANES * GROUP

    SUBS = 4

    def body(g, carry):
        ak, ap, thr = carry
        base = g * (SUBS * GB)
        xss = [[row_v[pl.ds(base + q * GB + k * LANES, LANES)]
                for k in range(GROUP)] for q in range(SUBS)]
        mq = [_tree(xs) for xs in xss]
        hit = jnp.any(_tree(mq) > thr)

        def sub(xs, m, sub_base, cry2):
            hs = jnp.any(m > cry2[2])

            def acc2(c3):
                ak, ap, _ = c3
                for k in range(GROUP):
                    idxv = iota + (sub_base + k * LANES)
                    ak, ap = _merge_top16(ak, ap, xs[k], idxv)
                return ak, ap, jnp.maximum(
                    jnp.broadcast_to(ak[0], (LANES,)), thr0)

            return lax.cond(hs, acc2, lambda c3: c3, cry2)

        def accept(cry):
            for h in range(SUBS // 2):
                def pair(c3, h=h):
                    def inner(c4, h=h):
                        c4 = sub(xss[2 * h], mq[2 * h],
                                 base + 2 * h * GB, c4)
                        return sub(xss[2 * h + 1], mq[2 * h + 1],
                                   base + (2 * h + 1) * GB, c4)

                    hp = jnp.any(
                        jnp.maximum(mq[2 * h], mq[2 * h + 1]) > c3[2])
                    return lax.cond(hp, inner, lambda c4: c4, c3)

                cry = pair(cry)
            return cry

        return lax.cond(hit, accept, lambda cry: cry, (ak, ap, thr))

    ak, ap, _ = lax.fori_loop(
        0, NGROUPS // SUBS, body,
        (neg, jnp.zeros((LANES,), jnp.int32), thr0))
    stage_f[...] = lax.rev(ak, (0,))
    stage_i[...] = lax.rev(ap, (0,))
    pltpu.sync_copy(stage_f, shared_v.at[pl.ds(s * LANES, LANES)])
    pltpu.sync_copy(stage_i, shared_w.at[pl.ds(s * LANES, LANES)])

    plsc.subcore_barrier()

    @pl.when(s == 0)
    def _tile0():
        pltpu.sync_copy(shared_v, vals_v)
        pltpu.sync_copy(shared_w, words_v)
        pltpu.make_async_copy(sum_hbm, sums_v, psem).wait()
        pltpu.make_async_copy(seq_hbm, seq_v, psem).wait()
        pltpu.make_async_copy(seqlp_hbm, seqlp_v, psem).wait()
        pltpu.make_async_copy(tvec_hbm, tvec_v, psem).wait()
        sumvec = sums_v[...]
        tvec = tvec_v[...]

        cols = []
        for cc in range(LANES):
            colv = plsc.load_gather(vals_v, [iota * LANES + cc])
            sk, sp = plsc.sort_key_val(colv + sumvec, iota + cc * LANES)
            cols.append((sk, sp))

        def merge_sorted(a, b):
            bk = lax.rev(b[0], (0,))
            bp = lax.rev(b[1], (0,))
            take = a[0] >= bk
            nk = jnp.where(take, a[0], bk)
            npay = jnp.where(take, a[1], bp)
            sk, sp = plsc.sort_key_val(nk, npay)
            return sk, sp

        while len(cols) > 1:
            cols = [merge_sorted(cols[i], cols[i + 1])
                    for i in range(0, len(cols), 2)]
        ak, ap = cols[0]
        fk = lax.rev(ak, (0,))
        fp = lax.rev(ap, (0,))
        r_sel = jnp.bitwise_and(fp, LANES - 1)
        c_sel = lax.shift_right_logical(fp, 4)
        words = plsc.load_gather(words_v, [r_sel * LANES + c_sel])
        wprob = plsc.load_gather(vals_v, [r_sel * LANES + c_sel])

        stage_f[...] = fk
        pltpu.async_copy(stage_f, out_sum_hbm, psem)

        for j in range(SEQ):
            jfull = jnp.full((LANES,), j, jnp.int32)
            am = jfull < tvec
            bm = jfull == tvec
            rows = jnp.where(am, r_sel, iota)
            colseq = plsc.load_gather(seq_v, [rows * SEQ + j])
            colseq = jnp.where(bm, words, colseq)
            plsc.store_scatter(oseq_v, [iota * SEQ + j], colseq)
            collp = plsc.load_gather(seqlp_v, [rows * SEQ + j])
            collp = jnp.where(bm, wprob, collp)
            plsc.store_scatter(oseqlp_v, [iota * SEQ + j], collp)
        pltpu.async_copy(oseq_v, out_seq_hbm, psem)
        pltpu.async_copy(oseqlp_v, out_seqlp_hbm, psem)

        sidx_v[pl.ds(0, LANES)] = r_sel
        sidx_v[pl.ds(LANES, LANES)] = r_sel + LANES
        pltpu.async_copy(state_hbm.at[sidx_v], srows_v, sem).wait()
        pltpu.sync_copy(srows_v, out_state_hbm)
        pltpu.make_async_copy(stage_f, out_sum_hbm, psem).wait()
        pltpu.make_async_copy(oseq_v, out_seq_hbm, psem).wait()
        pltpu.make_async_copy(oseqlp_v, out_seqlp_hbm, psem).wait()


_beam_step = pl.kernel(
    _beam_body,
    out_type=_OUT_TYPE,
    mesh=_mesh,
    scratch_types=_SCRATCH,
    compiler_params=pltpu.CompilerParams(needs_layout_passes=False),
)


def kernel(logprobsf, beam_seq, beam_seq_logprobs, beam_logprobs_sum, state,
           beam_size, t):
    del beam_size
    state2d = state.reshape(NLAYERS * BEAM, HID)
    tvec = jnp.broadcast_to(jnp.asarray(t, jnp.int32), (LANES,))
    oseq, oseqlp, osum, ostate = _beam_step(
        logprobsf, beam_seq.reshape(-1), beam_seq_logprobs.reshape(-1),
        beam_logprobs_sum, state2d, tvec)
    return (oseq.reshape(BEAM, SEQ), oseqlp.reshape(BEAM, SEQ), osum,
            ostate.reshape(NLAYERS, BEAM, HID))
